# baseline (speedup 1.0000x reference)
import functools

import jax
import jax.numpy as jnp
from jax import lax
from jax.experimental import pallas as pl
from jax.experimental.pallas import tpu as pltpu
from jax.experimental.pallas import tpu_sc as plsc

VOCAB = 100000
DIM = 128

NC = 2
NS = 16
NW = NC * NS

CHUNK = 128
NBUF2 = 4
LEAD = NBUF2 // 2


def _make_kernel(total_rows: int):
    rows_per_w = total_rows // NW
    cpw = rows_per_w // CHUNK
    assert cpw % NBUF2 == 0 and rows_per_w % CHUNK == 0

    mesh = plsc.VectorSubcoreMesh(core_axis_name="c", subcore_axis_name="s")

    scratch = [pltpu.VMEM((cpw, CHUNK), jnp.int32)]
    scratch += [pltpu.VMEM((CHUNK, DIM), jnp.float32) for _ in range(NBUF2)]
    scratch += [pltpu.SemaphoreType.DMA for _ in range(2 * NBUF2)]

    @functools.partial(
        pl.kernel,
        out_type=jax.ShapeDtypeStruct((total_rows // CHUNK, CHUNK, DIM),
                                      jnp.float32),
        mesh=mesh,
        scratch_types=scratch,
    )
    def emb_kernel(x_hbm, tab_hbm, out_hbm, idx_v, *rest):
        bufs = rest[:NBUF2]
        gsems = rest[NBUF2:2 * NBUF2]
        ssems = rest[2 * NBUF2:3 * NBUF2]

        wid = lax.axis_index("s") * NC + lax.axis_index("c")
        pltpu.sync_copy(x_hbm.at[wid], idx_v)

        chunk0 = wid * cpw

        def start_gather(b, g):
            pltpu.async_copy(tab_hbm.at[idx_v.at[g]], bufs[b], gsems[b])

        def wait_gather(b):
            pltpu.make_async_copy(tab_hbm.at[idx_v.at[0]], bufs[b],
                                  gsems[b]).wait()

        def start_scatter(b, g):
            pltpu.async_copy(bufs[b], out_hbm.at[chunk0 + g], ssems[b])

        def wait_scatter(b):
            pltpu.make_async_copy(bufs[b], out_hbm.at[0], ssems[b]).wait()

        def position(b, g, first_round, last_round):
            wait_gather(b)
            start_scatter(b, g)
            b2 = (b + LEAD) % NBUF2
            if not (first_round and b < LEAD):
                wait_scatter(b2)
            if not (last_round and b >= LEAD):
                start_gather(b2, g + LEAD)

        for b in range(LEAD):
            start_gather(b, b)

        for b in range(NBUF2):
            position(b, b, True, False)

        @pl.loop(NBUF2, cpw - NBUF2, step=NBUF2)
        def _(o):
            for b in range(NBUF2):
                position(b, o + b, False, False)

        for b in range(NBUF2):
            position(b, cpw - NBUF2 + b, False, True)

        for b in range(LEAD, NBUF2):
            wait_scatter(b)

    return emb_kernel


def kernel(x, emb):
    n, m = x.shape
    total = n * m
    idx = x.reshape(NW, total // (NW * CHUNK), CHUNK).astype(jnp.int32)
    out = _make_kernel(total)(idx, emb)
    return out.reshape(n, m, DIM)


# speedup vs baseline: 1.8336x; 1.8336x over previous
import functools

import jax
import jax.numpy as jnp
from jax import lax
from jax.experimental import pallas as pl
from jax.experimental.pallas import tpu as pltpu
from jax.experimental.pallas import tpu_sc as plsc

VOCAB = 100000
DIM = 128

NC = 2
NS = 16
NW = NC * NS

ROWS_PER_CHUNK = 4
NBUF2 = 4
LEAD = NBUF2 // 2
SPLITS = ((0, 104), (104, 96))


def _make_kernel(n_rows: int, m: int):
    ipw = n_rows // NW
    cpw = ipw // ROWS_PER_CHUNK
    cm = ROWS_PER_CHUNK * m
    assert cpw % NBUF2 == 0 and ipw % ROWS_PER_CHUNK == 0 and cm % 8 == 0

    mesh = plsc.VectorSubcoreMesh(core_axis_name="c", subcore_axis_name="s")

    scratch = [pltpu.VMEM((ipw * m,), jnp.int32)]
    scratch += [pltpu.VMEM((cm, DIM), jnp.float32) for _ in range(NBUF2)]
    scratch += [pltpu.SemaphoreType.DMA for _ in range(2 * NBUF2)]

    @functools.partial(
        pl.kernel,
        out_type=jax.ShapeDtypeStruct((n_rows, m, DIM), jnp.float32),
        mesh=mesh,
        scratch_types=scratch,
    )
    def emb_kernel(x_hbm, tab_hbm, out_hbm, idx_v, *rest):
        bufs = rest[:NBUF2]
        gsems = rest[NBUF2:2 * NBUF2]
        ssems = rest[2 * NBUF2:3 * NBUF2]

        wid = lax.axis_index("s") * NC + lax.axis_index("c")
        pltpu.sync_copy(x_hbm.at[wid], idx_v)

        row0 = wid * ipw

        def start_gather(b, g):
            for off, ln in SPLITS:
                pltpu.async_copy(
                    tab_hbm.at[idx_v.at[pl.ds(g * cm + off, ln)]],
                    bufs[b].at[pl.ds(off, ln)], gsems[b])

        def wait_gather(b):
            for off, ln in SPLITS:
                pltpu.make_async_copy(
                    tab_hbm.at[idx_v.at[pl.ds(0, ln)]],
                    bufs[b].at[pl.ds(off, ln)], gsems[b]).wait()

        def start_scatter(b, g):
            i0 = row0 + g * ROWS_PER_CHUNK
            for r in range(ROWS_PER_CHUNK):
                pltpu.async_copy(bufs[b].at[pl.ds(r * m, m)],
                                 out_hbm.at[i0 + r], ssems[b])

        def wait_scatter(b):
            for r in range(ROWS_PER_CHUNK):
                pltpu.make_async_copy(bufs[b].at[pl.ds(r * m, m)],
                                      out_hbm.at[0], ssems[b]).wait()

        def position(b, g, first_round, last_round):
            wait_gather(b)
            start_scatter(b, g)
            b2 = (b + LEAD) % NBUF2
            if not (first_round and b < LEAD):
                wait_scatter(b2)
            if not (last_round and b >= LEAD):
                start_gather(b2, g + LEAD)

        for b in range(LEAD):
            start_gather(b, b)

        for b in range(NBUF2):
            position(b, b, True, False)

        @pl.loop(NBUF2, cpw - NBUF2, step=NBUF2)
        def _(o):
            for b in range(NBUF2):
                position(b, o + b, False, False)

        for b in range(NBUF2):
            position(b, cpw - NBUF2 + b, False, True)

        for b in range(LEAD, NBUF2):
            wait_scatter(b)

    return emb_kernel


def kernel(x, emb):
    n, m = x.shape
    idx = x.reshape(NW, (n // NW) * m).astype(jnp.int32)
    return _make_kernel(n, m)(idx, emb)
